# baseline (speedup 1.0000x reference)
import functools

import jax
import jax.numpy as jnp
from jax import lax
from jax.experimental import pallas as pl
from jax.experimental.pallas import tpu as pltpu
from jax.experimental.pallas import tpu_sc as plsc

N = 10000
E = 160000
D_IN = 256
D_OUT = 256
EPS = 1e-5

NC = 2
NS = 16
NPAD = 10240
RPT = NPAD // NS
CHUNK = 128
EPT = 5120
EPAD = EPT * NS
NCHUNK = EPT // CHUNK
NPAIR = NCHUNK // 2
LANES = 16


def _sc_scatter_body(tab_hbm, src_hbm, dst_hbm, ah_hbm, deg_hbm,
                     srcall_v, dst_a, dst_b, rows_a, rows_b, ones_v, zer_v,
                     acc_s, dega_s, sem_a, sem_b):
    c = lax.axis_index("c")
    s = lax.axis_index("s")

    z32 = jnp.zeros((2 * LANES,), jnp.bfloat16)
    z16 = jnp.zeros((LANES,), jnp.float32)
    o16 = jnp.ones((LANES,), jnp.float32)

    def fill_rows(r, _):
        for t in range(2):
            for j in range(128 // (2 * LANES)):
                rows_a[r, t, pl.ds(j * 2 * LANES, 2 * LANES)] = z32
        return 0
    lax.fori_loop(0, CHUNK, fill_rows, 0)

    def fill_ones(j, _):
        ones_v[pl.ds(j * LANES, LANES)] = o16
        return 0
    lax.fori_loop(0, CHUNK // LANES, fill_ones, 0)

    def fill_zer(j, _):
        zer_v[pl.ds(j * LANES, LANES)] = z16
        return 0
    lax.fori_loop(0, RPT // LANES, fill_zer, 0)

    rbase = s * RPT
    for j in range(RPT // CHUNK):
        pltpu.sync_copy(rows_a, acc_s.at[pl.ds(rbase + j * CHUNK, CHUNK)])
    pltpu.sync_copy(zer_v, dega_s.at[pl.ds(rbase, RPT)])

    ebase = c * EPAD + s * EPT
    pltpu.sync_copy(src_hbm.at[pl.ds(ebase, EPT)], srcall_v)
    plsc.subcore_barrier()

    def issue(i, buf, dbuf, sem):
        pltpu.async_copy(
            tab_hbm.at[srcall_v.at[pl.ds(i * CHUNK, CHUNK)]], buf, sem)
        pltpu.async_copy(dst_hbm.at[pl.ds(ebase + i * CHUNK, CHUNK)],
                         dbuf, sem)

    def drain(buf, dbuf, sem):
        pltpu.make_async_copy(tab_hbm.at[srcall_v.at[pl.ds(0, CHUNK)]],
                              buf, sem).wait()
        pltpu.make_async_copy(dst_hbm.at[pl.ds(0, CHUNK)], dbuf, sem).wait()

    def scat(i, buf, dbuf):
        pltpu.sync_copy(buf, acc_s.at[dbuf], add=True)
        pltpu.sync_copy(ones_v, dega_s.at[dbuf], add=True)

    issue(0, rows_a, dst_a, sem_a)

    def pair(k, _):
        issue(2 * k + 1, rows_b, dst_b, sem_b)
        drain(rows_a, dst_a, sem_a)
        scat(2 * k, rows_a, dst_a)

        @pl.when(k < NPAIR - 1)
        def _():
            issue(2 * k + 2, rows_a, dst_a, sem_a)

        drain(rows_b, dst_b, sem_b)
        scat(2 * k + 1, rows_b, dst_b)
        return 0

    lax.fori_loop(0, NPAIR, pair, 0)
    plsc.subcore_barrier()

    pltpu.sync_copy(acc_s.at[pl.ds(rbase, RPT)],
                    ah_hbm.at[c, pl.ds(rbase, RPT)])
    pltpu.sync_copy(dega_s.at[pl.ds(rbase, RPT)],
                    deg_hbm.at[c, pl.ds(rbase, RPT)])


_sc_scatter = functools.partial(
    pl.kernel,
    out_type=[
        jax.ShapeDtypeStruct((NC, NPAD, 2, 128), jnp.bfloat16),
        jax.ShapeDtypeStruct((NC, NPAD), jnp.float32),
    ],
    mesh=plsc.VectorSubcoreMesh(core_axis_name="c", subcore_axis_name="s"),
    compiler_params=pltpu.CompilerParams(use_tc_tiling_on_sc=False),
    scratch_types=[
        pltpu.VMEM((EPT,), jnp.int32),
        pltpu.VMEM((CHUNK,), jnp.int32),
        pltpu.VMEM((CHUNK,), jnp.int32),
        pltpu.VMEM((CHUNK, 2, 128), jnp.bfloat16),
        pltpu.VMEM((CHUNK, 2, 128), jnp.bfloat16),
        pltpu.VMEM((CHUNK,), jnp.float32),
        pltpu.VMEM((RPT,), jnp.float32),
        pltpu.VMEM_SHARED((NPAD, 2, 128), jnp.bfloat16),
        pltpu.VMEM_SHARED((NPAD,), jnp.float32),
        pltpu.SemaphoreType.DMA,
        pltpu.SemaphoreType.DMA,
    ],
)(_sc_scatter_body)


def _tc_dense_body(h_ref, a0_ref, a1_ref, deg_ref, w_ref, b_ref, g_ref,
                   be_ref, o_ref):
    d = deg_ref[...]
    deg = d[:, :1] + d[:, 1:2]
    norm = jnp.where(deg > 0, 1.0 / jnp.maximum(deg, 1.0), 0.0)
    a = (a0_ref[...].astype(jnp.float32) +
         a1_ref[...].astype(jnp.float32)) * norm
    w = w_ref[...]
    dn = (((1,), (1,)), ((), ()))
    out = lax.dot_general(h_ref[...], w[:, :D_IN], dn,
                          preferred_element_type=jnp.float32)
    out += lax.dot_general(a, w[:, D_IN:], dn,
                           preferred_element_type=jnp.float32)
    out += b_ref[...]
    mean = jnp.mean(out, axis=-1, keepdims=True)
    cent = out - mean
    var = jnp.mean(cent * cent, axis=-1, keepdims=True)
    out = cent / jnp.sqrt(var + EPS) * g_ref[...] + be_ref[...]
    o_ref[...] = jnp.maximum(out, 0.0)


ROWS_BLK = 400
GRID = N // ROWS_BLK

_tc_dense = pl.pallas_call(
    _tc_dense_body,
    grid=(GRID,),
    in_specs=[
        pl.BlockSpec((ROWS_BLK, D_IN), lambda i: (i, 0)),
        pl.BlockSpec((ROWS_BLK, D_IN), lambda i: (i, 0)),
        pl.BlockSpec((ROWS_BLK, D_IN), lambda i: (i, 0)),
        pl.BlockSpec((ROWS_BLK, NC), lambda i: (i, 0)),
        pl.BlockSpec((D_OUT, 2 * D_IN), lambda i: (0, 0)),
        pl.BlockSpec((1, D_OUT), lambda i: (0, 0)),
        pl.BlockSpec((1, D_OUT), lambda i: (0, 0)),
        pl.BlockSpec((1, D_OUT), lambda i: (0, 0)),
    ],
    out_specs=pl.BlockSpec((ROWS_BLK, D_OUT), lambda i: (i, 0)),
    out_shape=jax.ShapeDtypeStruct((N, D_OUT), jnp.float32),
)


def kernel(h, edge_index, W, b, gamma, beta):
    src = edge_index[0]
    dst = edge_index[1]

    npad_e = NC * EPAD - E
    src_p = jnp.concatenate([src, jnp.zeros((npad_e,), jnp.int32)])
    dst_p = jnp.concatenate([dst, jnp.full((npad_e,), N, jnp.int32)])
    tab = h.astype(jnp.bfloat16).reshape(N, 2, 128)

    ah2, deg2 = _sc_scatter(tab, src_p, dst_p)
    ah2 = ah2.reshape(NC, NPAD, D_IN)

    return _tc_dense(h, ah2[0], ah2[1], deg2.T, W, b.reshape(1, D_OUT),
                     gamma.reshape(1, D_OUT), beta.reshape(1, D_OUT))


# speedup vs baseline: 1.0183x; 1.0183x over previous
import functools

import jax
import jax.numpy as jnp
from jax import lax
from jax.experimental import pallas as pl
from jax.experimental.pallas import tpu as pltpu
from jax.experimental.pallas import tpu_sc as plsc

N = 10000
E = 160000
D_IN = 256
D_OUT = 256
DH = D_IN // 2
EPS = 1e-5

NC = 2
NS = 16
NPAD = 10240
RPT = NPAD // NS
CHUNK = 128
EPT = 10240
EPAD = EPT * NS
NCHUNK = EPT // CHUNK
NPAIR = NCHUNK // 2
LANES = 16


def _sc_scatter_body(tab_hbm, src_hbm, dst_hbm, ah_hbm, deg_hbm,
                     srcall_v, dst_a, dst_b, rows_a, rows_b, ones_v, zer_v,
                     acc_s, dega_s, sem_a, sem_b):
    c = lax.axis_index("c")
    s = lax.axis_index("s")

    z16 = jnp.zeros((LANES,), jnp.float32)
    o16 = jnp.ones((LANES,), jnp.float32)

    def fill_rows(r, _):
        for j in range(DH // LANES):
            rows_a[r, pl.ds(j * LANES, LANES)] = z16
        return 0
    lax.fori_loop(0, CHUNK, fill_rows, 0)

    def fill_ones(j, _):
        ones_v[pl.ds(j * LANES, LANES)] = o16
        return 0
    lax.fori_loop(0, CHUNK // LANES, fill_ones, 0)

    def fill_zer(j, _):
        zer_v[pl.ds(j * LANES, LANES)] = z16
        return 0
    lax.fori_loop(0, RPT // LANES, fill_zer, 0)

    rbase = s * RPT
    for j in range(RPT // CHUNK):
        pltpu.sync_copy(rows_a, acc_s.at[pl.ds(rbase + j * CHUNK, CHUNK)])
    pltpu.sync_copy(zer_v, dega_s.at[pl.ds(rbase, RPT)])

    pltpu.sync_copy(src_hbm.at[pl.ds(s * EPT, EPT)], srcall_v)
    plsc.subcore_barrier()

    tab_hbm = tab_hbm.at[:, pl.ds(c * DH, DH)]

    ebase = s * EPT

    def issue(i, buf, dbuf, sem):
        pltpu.async_copy(
            tab_hbm.at[srcall_v.at[pl.ds(i * CHUNK, CHUNK)]], buf, sem)
        pltpu.async_copy(dst_hbm.at[pl.ds(ebase + i * CHUNK, CHUNK)],
                         dbuf, sem)

    def drain(buf, dbuf, sem):
        pltpu.make_async_copy(tab_hbm.at[srcall_v.at[pl.ds(0, CHUNK)]],
                              buf, sem).wait()
        pltpu.make_async_copy(dst_hbm.at[pl.ds(0, CHUNK)], dbuf, sem).wait()

    def owned(i):
        return (i < NCHUNK // 2) == (c == 0)

    def scat(i, buf, dbuf):
        pltpu.sync_copy(buf, acc_s.at[dbuf], add=True)

        @pl.when(owned(i))
        def _():
            pltpu.sync_copy(ones_v, dega_s.at[dbuf], add=True)

    issue(0, rows_a, dst_a, sem_a)

    def pair(k, _):
        issue(2 * k + 1, rows_b, dst_b, sem_b)
        drain(rows_a, dst_a, sem_a)
        scat(2 * k, rows_a, dst_a)

        @pl.when(k < NPAIR - 1)
        def _():
            issue(2 * k + 2, rows_a, dst_a, sem_a)

        drain(rows_b, dst_b, sem_b)
        scat(2 * k + 1, rows_b, dst_b)
        return 0

    lax.fori_loop(0, NPAIR, pair, 0)
    plsc.subcore_barrier()

    pltpu.sync_copy(acc_s.at[pl.ds(rbase, RPT)],
                    ah_hbm.at[c, pl.ds(rbase, RPT)])
    pltpu.sync_copy(dega_s.at[pl.ds(rbase, RPT)],
                    deg_hbm.at[c, pl.ds(rbase, RPT)])


_sc_scatter = functools.partial(
    pl.kernel,
    out_type=[
        jax.ShapeDtypeStruct((NC, NPAD, DH), jnp.float32),
        jax.ShapeDtypeStruct((NC, NPAD), jnp.float32),
    ],
    mesh=plsc.VectorSubcoreMesh(core_axis_name="c", subcore_axis_name="s"),
    scratch_types=[
        pltpu.VMEM((EPT,), jnp.int32),
        pltpu.VMEM((CHUNK,), jnp.int32),
        pltpu.VMEM((CHUNK,), jnp.int32),
        pltpu.VMEM((CHUNK, DH), jnp.float32),
        pltpu.VMEM((CHUNK, DH), jnp.float32),
        pltpu.VMEM((CHUNK,), jnp.float32),
        pltpu.VMEM((RPT,), jnp.float32),
        pltpu.VMEM_SHARED((NPAD, DH), jnp.float32),
        pltpu.VMEM_SHARED((NPAD,), jnp.float32),
        pltpu.SemaphoreType.DMA,
        pltpu.SemaphoreType.DMA,
    ],
)(_sc_scatter_body)


ROWS_BLK = 400
GRID = N // ROWS_BLK


def _tc_hpart_body(h_ref, w_ref, o_ref):
    dn = (((1,), (1,)), ((), ()))
    o_ref[...] = lax.dot_general(h_ref[...], w_ref[...][:, :D_IN], dn,
                                 preferred_element_type=jnp.float32)


_tc_hpart = pl.pallas_call(
    _tc_hpart_body,
    grid=(GRID,),
    in_specs=[
        pl.BlockSpec((ROWS_BLK, D_IN), lambda i: (i, 0)),
        pl.BlockSpec((D_OUT, 2 * D_IN), lambda i: (0, 0)),
    ],
    out_specs=pl.BlockSpec((ROWS_BLK, D_OUT), lambda i: (i, 0)),
    out_shape=jax.ShapeDtypeStruct((N, D_OUT), jnp.float32),
)


def _tc_dense_body(p_ref, a0_ref, a1_ref, deg_ref, w_ref, b_ref, g_ref,
                   be_ref, o_ref):
    d = deg_ref[...]
    deg = d[:, :1] + d[:, 1:2]
    norm = jnp.where(deg > 0, 1.0 / jnp.maximum(deg, 1.0), 0.0)
    w = w_ref[...]
    dn = (((1,), (1,)), ((), ()))
    out = p_ref[...]
    out += lax.dot_general(a0_ref[...] * norm, w[:, D_IN:D_IN + DH], dn,
                           preferred_element_type=jnp.float32)
    out += lax.dot_general(a1_ref[...] * norm, w[:, D_IN + DH:], dn,
                           preferred_element_type=jnp.float32)
    out += b_ref[...]
    mean = jnp.mean(out, axis=-1, keepdims=True)
    cent = out - mean
    var = jnp.mean(cent * cent, axis=-1, keepdims=True)
    out = cent / jnp.sqrt(var + EPS) * g_ref[...] + be_ref[...]
    o_ref[...] = jnp.maximum(out, 0.0)


_tc_dense = pl.pallas_call(
    _tc_dense_body,
    grid=(GRID,),
    in_specs=[
        pl.BlockSpec((ROWS_BLK, D_OUT), lambda i: (i, 0)),
        pl.BlockSpec((ROWS_BLK, DH), lambda i: (i, 0)),
        pl.BlockSpec((ROWS_BLK, DH), lambda i: (i, 0)),
        pl.BlockSpec((ROWS_BLK, NC), lambda i: (i, 0)),
        pl.BlockSpec((D_OUT, 2 * D_IN), lambda i: (0, 0)),
        pl.BlockSpec((1, D_OUT), lambda i: (0, 0)),
        pl.BlockSpec((1, D_OUT), lambda i: (0, 0)),
        pl.BlockSpec((1, D_OUT), lambda i: (0, 0)),
    ],
    out_specs=pl.BlockSpec((ROWS_BLK, D_OUT), lambda i: (i, 0)),
    out_shape=jax.ShapeDtypeStruct((N, D_OUT), jnp.float32),
)


def kernel(h, edge_index, W, b, gamma, beta):
    src = edge_index[0]
    dst = edge_index[1]

    npad_e = EPAD - E
    src_p = jnp.concatenate([src, jnp.zeros((npad_e,), jnp.int32)])
    dst_p = jnp.concatenate([dst, jnp.full((npad_e,), N, jnp.int32)])

    ah, deg2 = _sc_scatter(h, src_p, dst_p)
    p = _tc_hpart(h, W)

    return _tc_dense(p, ah[0], ah[1], deg2.T, W, b.reshape(1, D_OUT),
                     gamma.reshape(1, D_OUT), beta.reshape(1, D_OUT))
